# Initial kernel scaffold; baseline (speedup 1.0000x reference)
import jax
import jax.numpy as jnp
from jax.experimental import pallas as pl


def kernel(x, indices, W_edge, b_edge, g_edge, beta_edge, W1, b1, g1, beta1, W2, b2, g2, beta2, W3, b3, g3, beta3, Wf, bf):
    raise NotImplementedError("write your pallas kernel here")


# speedup vs baseline: 49.0877x; 49.0877x over previous
import functools

import jax
import jax.numpy as jnp
from jax import lax
from jax.experimental import pallas as pl
from jax.experimental.pallas import tpu as pltpu
from jax.experimental.pallas import tpu_sc as plsc

B, N, K = 16, 2048, 20
TOTAL = B * N
NC, NS = 2, 16
NW = NC * NS
PPW = TOTAL // NW
G = 32
ROWS = G * K
IDXW = 128
NDMA = ROWS // IDXW
NCHUNK = PPW // G


def _make_pool(C, fuse_u):
    mesh = plsc.VectorSubcoreMesh(
        core_axis_name="c", subcore_axis_name="s",
        num_cores=NC, num_subcores=NS)
    scratch = [
        pltpu.VMEM((NDMA, IDXW), jnp.int32),
        pltpu.VMEM((ROWS, C), jnp.float32),
        pltpu.VMEM((G, C), jnp.float32),
    ]
    if fuse_u:
        scratch.append(pltpu.VMEM((G, C), jnp.float32))
    scratch.append(pltpu.SemaphoreType.DMA)

    def body(table_hbm, gidx_hbm, *rest):
        if fuse_u:
            u_hbm, out_hbm, idx_v, rows_v, out_v, u_v, sem = rest
        else:
            out_hbm, idx_v, rows_v, out_v, sem = rest
        wid = lax.axis_index("s") * NC + lax.axis_index("c")
        base = wid * PPW

        def chunk(i, carry):
            cbase = base + i * G
            pltpu.sync_copy(gidx_hbm.at[wid * NCHUNK + i], idx_v)
            cps = [
                pltpu.async_copy(table_hbm.at[idx_v.at[j]],
                                 rows_v.at[pl.ds(j * IDXW, IDXW)], sem)
                for j in range(NDMA)
            ]
            if fuse_u:
                pltpu.sync_copy(u_hbm.at[pl.ds(cbase, G)], u_v)
            for cp in cps:
                cp.wait()

            def point(g, c2):
                r0 = g * K
                for co in range(C // 16):
                    sl = pl.ds(co * 16, 16)
                    acc = rows_v[r0, sl]
                    for kk in range(1, K):
                        acc = jnp.maximum(acc, rows_v[r0 + kk, sl])
                    if fuse_u:
                        t = acc + u_v[g, sl]
                        acc = jnp.maximum(t, 0.2 * t)
                    out_v[g, sl] = acc
                return c2

            lax.fori_loop(0, G, point, 0)
            pltpu.sync_copy(out_v, out_hbm.at[pl.ds(cbase, G)])
            return carry

        lax.fori_loop(0, NCHUNK, chunk, 0)

    return pl.kernel(
        body,
        out_type=jax.ShapeDtypeStruct((TOTAL, C), jnp.float32),
        mesh=mesh,
        scratch_types=scratch,
        compiler_params=pltpu.CompilerParams(use_tc_tiling_on_sc=False),
    )


def _pu_kernel(x_ref, wp_ref, wu_ref, bu_ref, p_ref, u_ref):
    xb = x_ref[...]
    p = (xb[:, 0:1] * wp_ref[0:1, :] + xb[:, 1:2] * wp_ref[1:2, :]
         + xb[:, 2:3] * wp_ref[2:3, :])
    u = (xb[:, 0:1] * wu_ref[0:1, :] + xb[:, 1:2] * wu_ref[1:2, :]
         + xb[:, 2:3] * wu_ref[2:3, :]) + bu_ref[...]
    p_ref[...] = p
    u_ref[...] = u


def _mm_kernel(x_ref, wt_ref, b_ref, o_ref):
    z = jnp.dot(x_ref[...], wt_ref[...],
                preferred_element_type=jnp.float32) + b_ref[...]
    o_ref[...] = jnp.maximum(z, 0.2 * z)


def _final_kernel(h0_ref, h1_ref, h2_ref, h3_ref, wt_ref, b_ref, o_ref):
    cat = jnp.concatenate(
        [h0_ref[...], h1_ref[...], h2_ref[...], h3_ref[...]], axis=1)
    z = jnp.dot(cat, wt_ref[...],
                preferred_element_type=jnp.float32) + b_ref[...]
    z = jnp.maximum(z, 0.2 * z)
    b = pl.program_id(0)
    o_ref[pl.ds(b, 1), :] = jnp.max(z, axis=0, keepdims=True)


def _fold(W, b, g, beta):
    return g[:, None] * W, g * b + beta


def kernel(x, indices, W_edge, b_edge, g_edge, beta_edge,
           W1, b1, g1, beta1, W2, b2, g2, beta2, W3, b3, g3, beta3, Wf, bf):
    f32 = jnp.float32
    We, be = _fold(W_edge, b_edge, g_edge, beta_edge)
    Wc, Wd = We[:, :3], We[:, 3:]
    W1f, b1f = _fold(W1, b1, g1, beta1)
    W2f, b2f = _fold(W2, b2, g2, beta2)
    W3f, b3f = _fold(W3, b3, g3, beta3)

    xf = x.reshape(TOTAL, 3).astype(f32)
    gidx = (indices.astype(jnp.int32)
            + (jnp.arange(B, dtype=jnp.int32) * N)[:, None, None])
    gidx = gidx.reshape(NW * NCHUNK, NDMA, IDXW)

    R = 2048
    grid = (TOTAL // R,)
    p, u = pl.pallas_call(
        _pu_kernel,
        grid=grid,
        in_specs=[
            pl.BlockSpec((R, 3), lambda i: (i, 0)),
            pl.BlockSpec((3, 64), lambda i: (0, 0)),
            pl.BlockSpec((3, 64), lambda i: (0, 0)),
            pl.BlockSpec((1, 64), lambda i: (0, 0)),
        ],
        out_specs=[pl.BlockSpec((R, 64), lambda i: (i, 0)),
                   pl.BlockSpec((R, 64), lambda i: (i, 0))],
        out_shape=[jax.ShapeDtypeStruct((TOTAL, 64), f32),
                   jax.ShapeDtypeStruct((TOTAL, 64), f32)],
    )(xf, Wd.T, (Wc - Wd).T, be.reshape(1, 64))

    h0 = _make_pool(64, fuse_u=True)(p, gidx, u)

    def mm(h, Wl, bl, Cin, Cout):
        return pl.pallas_call(
            _mm_kernel,
            grid=grid,
            in_specs=[
                pl.BlockSpec((R, Cin), lambda i: (i, 0)),
                pl.BlockSpec((Cin, Cout), lambda i: (0, 0)),
                pl.BlockSpec((1, Cout), lambda i: (0, 0)),
            ],
            out_specs=pl.BlockSpec((R, Cout), lambda i: (i, 0)),
            out_shape=jax.ShapeDtypeStruct((TOTAL, Cout), f32),
        )(h, Wl.T, bl.reshape(1, Cout))

    pool64 = _make_pool(64, fuse_u=False)
    h1 = mm(pool64(h0, gidx), W1f, b1f, 64, 64)
    h2 = mm(pool64(h1, gidx), W2f, b2f, 64, 128)
    h3 = mm(_make_pool(128, fuse_u=False)(h2, gidx), W3f, b3f, 128, 256)

    out = pl.pallas_call(
        _final_kernel,
        grid=(B,),
        in_specs=[
            pl.BlockSpec((N, 64), lambda b: (b, 0)),
            pl.BlockSpec((N, 64), lambda b: (b, 0)),
            pl.BlockSpec((N, 128), lambda b: (b, 0)),
            pl.BlockSpec((N, 256), lambda b: (b, 0)),
            pl.BlockSpec((512, 512), lambda b: (0, 0)),
            pl.BlockSpec((1, 512), lambda b: (0, 0)),
        ],
        out_specs=pl.BlockSpec((B, 512), lambda b: (0, 0)),
        out_shape=jax.ShapeDtypeStruct((B, 512), f32),
    )(h0, h1, h2, h3, Wf.T, bf.reshape(1, 512))
    return out


# speedup vs baseline: 67.2667x; 1.3703x over previous
import functools

import jax
import jax.numpy as jnp
from jax import lax
from jax.experimental import pallas as pl
from jax.experimental.pallas import tpu as pltpu
from jax.experimental.pallas import tpu_sc as plsc

B, N, K = 16, 2048, 20
TOTAL = B * N
NC, NS = 2, 16
NW = NC * NS
PPW = TOTAL // NW


def _make_pool(C, fuse_u, G, IDXW):
    ROWS = G * K
    NDMA = ROWS // IDXW
    NCHUNK = PPW // G
    assert ROWS % IDXW == 0 and NCHUNK % 2 == 0
    mesh = plsc.VectorSubcoreMesh(
        core_axis_name="c", subcore_axis_name="s",
        num_cores=NC, num_subcores=NS)
    scratch = [
        [pltpu.VMEM((NDMA, IDXW), jnp.int32) for _ in range(2)],
        [pltpu.VMEM((ROWS, C), jnp.float32) for _ in range(2)],
        pltpu.VMEM((G, C), jnp.float32),
    ]
    if fuse_u:
        scratch.append([pltpu.VMEM((G, C), jnp.float32) for _ in range(2)])
    scratch.append([pltpu.SemaphoreType.DMA for _ in range(2)])

    def body(table_hbm, gidx_hbm, *rest):
        if fuse_u:
            u_hbm, out_hbm, idx_b, rows_b, out_v, u_b, sem_b = rest
        else:
            out_hbm, idx_b, rows_b, out_v, sem_b = rest
        wid = lax.axis_index("s") * NC + lax.axis_index("c")
        base = wid * PPW

        def fire(c, s):
            pltpu.sync_copy(gidx_hbm.at[wid * NCHUNK + c], idx_b[s])
            for j in range(NDMA):
                pltpu.async_copy(table_hbm.at[idx_b[s].at[j]],
                                 rows_b[s].at[pl.ds(j * IDXW, IDXW)],
                                 sem_b[s])
            if fuse_u:
                pltpu.sync_copy(u_hbm.at[pl.ds(base + c * G, G)], u_b[s])

        def drain_compute(c, s):
            for j in range(NDMA):
                pltpu.make_async_copy(
                    table_hbm.at[idx_b[s].at[j]],
                    rows_b[s].at[pl.ds(j * IDXW, IDXW)],
                    sem_b[s]).wait()
            rows_v = rows_b[s]

            def point(g, c2):
                r0 = g * K
                for co in range(C // 16):
                    sl = pl.ds(co * 16, 16)
                    acc = rows_v[r0, sl]
                    for kk in range(1, K):
                        acc = jnp.maximum(acc, rows_v[r0 + kk, sl])
                    if fuse_u:
                        t = acc + u_b[s][g, sl]
                        acc = jnp.maximum(t, 0.2 * t)
                    out_v[g, sl] = acc
                return c2

            lax.fori_loop(0, G, point, 0)
            pltpu.sync_copy(out_v, out_hbm.at[pl.ds(base + c * G, G)])

        fire(0, 0)

        def pair(t, carry):
            i0 = 2 * t
            fire(i0 + 1, 1)
            drain_compute(i0, 0)

            @pl.when(i0 + 2 < NCHUNK)
            def _():
                fire(i0 + 2, 0)

            drain_compute(i0 + 1, 1)
            return carry

        lax.fori_loop(0, NCHUNK // 2, pair, 0)

    return pl.kernel(
        body,
        out_type=jax.ShapeDtypeStruct((TOTAL, C), jnp.float32),
        mesh=mesh,
        scratch_types=scratch,
        compiler_params=pltpu.CompilerParams(use_tc_tiling_on_sc=False),
    )


def _pu_kernel(x_ref, wp_ref, wu_ref, bu_ref, p_ref, u_ref):
    xb = x_ref[...]
    p = (xb[:, 0:1] * wp_ref[0:1, :] + xb[:, 1:2] * wp_ref[1:2, :]
         + xb[:, 2:3] * wp_ref[2:3, :])
    u = (xb[:, 0:1] * wu_ref[0:1, :] + xb[:, 1:2] * wu_ref[1:2, :]
         + xb[:, 2:3] * wu_ref[2:3, :]) + bu_ref[...]
    p_ref[...] = p
    u_ref[...] = u


def _mm_kernel(x_ref, wt_ref, b_ref, o_ref):
    z = jnp.dot(x_ref[...], wt_ref[...],
                preferred_element_type=jnp.float32) + b_ref[...]
    o_ref[...] = jnp.maximum(z, 0.2 * z)


def _final_kernel(h0_ref, h1_ref, h2_ref, h3_ref, wt_ref, b_ref, o_ref):
    cat = jnp.concatenate(
        [h0_ref[...], h1_ref[...], h2_ref[...], h3_ref[...]], axis=1)
    z = jnp.dot(cat, wt_ref[...],
                preferred_element_type=jnp.float32) + b_ref[...]
    z = jnp.maximum(z, 0.2 * z)
    b = pl.program_id(0)
    o_ref[pl.ds(b, 1), :] = jnp.max(z, axis=0, keepdims=True)


def _fold(W, b, g, beta):
    return g[:, None] * W, g * b + beta


def kernel(x, indices, W_edge, b_edge, g_edge, beta_edge,
           W1, b1, g1, beta1, W2, b2, g2, beta2, W3, b3, g3, beta3, Wf, bf):
    f32 = jnp.float32
    We, be = _fold(W_edge, b_edge, g_edge, beta_edge)
    Wc, Wd = We[:, :3], We[:, 3:]
    W1f, b1f = _fold(W1, b1, g1, beta1)
    W2f, b2f = _fold(W2, b2, g2, beta2)
    W3f, b3f = _fold(W3, b3, g3, beta3)

    xf = x.reshape(TOTAL, 3).astype(f32)
    gidx = (indices.astype(jnp.int32)
            + (jnp.arange(B, dtype=jnp.int32) * N)[:, None, None])
    gidx64 = gidx.reshape(NW * (PPW // 32), 5, 128)
    gidx128 = gidx.reshape(NW * (PPW // 16), 5, 64)

    R = 2048
    grid = (TOTAL // R,)
    p, u = pl.pallas_call(
        _pu_kernel,
        grid=grid,
        in_specs=[
            pl.BlockSpec((R, 3), lambda i: (i, 0)),
            pl.BlockSpec((3, 64), lambda i: (0, 0)),
            pl.BlockSpec((3, 64), lambda i: (0, 0)),
            pl.BlockSpec((1, 64), lambda i: (0, 0)),
        ],
        out_specs=[pl.BlockSpec((R, 64), lambda i: (i, 0)),
                   pl.BlockSpec((R, 64), lambda i: (i, 0))],
        out_shape=[jax.ShapeDtypeStruct((TOTAL, 64), f32),
                   jax.ShapeDtypeStruct((TOTAL, 64), f32)],
    )(xf, Wd.T, (Wc - Wd).T, be.reshape(1, 64))

    h0 = _make_pool(64, True, 32, 128)(p, gidx64, u)

    def mm(h, Wl, bl, Cin, Cout):
        return pl.pallas_call(
            _mm_kernel,
            grid=grid,
            in_specs=[
                pl.BlockSpec((R, Cin), lambda i: (i, 0)),
                pl.BlockSpec((Cin, Cout), lambda i: (0, 0)),
                pl.BlockSpec((1, Cout), lambda i: (0, 0)),
            ],
            out_specs=pl.BlockSpec((R, Cout), lambda i: (i, 0)),
            out_shape=jax.ShapeDtypeStruct((TOTAL, Cout), f32),
        )(h, Wl.T, bl.reshape(1, Cout))

    pool64 = _make_pool(64, False, 32, 128)
    h1 = mm(pool64(h0, gidx64), W1f, b1f, 64, 64)
    h2 = mm(pool64(h1, gidx64), W2f, b2f, 64, 128)
    h3 = mm(_make_pool(128, False, 16, 64)(h2, gidx128), W3f, b3f, 128, 256)

    out = pl.pallas_call(
        _final_kernel,
        grid=(B,),
        in_specs=[
            pl.BlockSpec((N, 64), lambda b: (b, 0)),
            pl.BlockSpec((N, 64), lambda b: (b, 0)),
            pl.BlockSpec((N, 128), lambda b: (b, 0)),
            pl.BlockSpec((N, 256), lambda b: (b, 0)),
            pl.BlockSpec((512, 512), lambda b: (0, 0)),
            pl.BlockSpec((1, 512), lambda b: (0, 0)),
        ],
        out_specs=pl.BlockSpec((B, 512), lambda b: (0, 0)),
        out_shape=jax.ShapeDtypeStruct((B, 512), f32),
    )(h0, h1, h2, h3, Wf.T, bf.reshape(1, 512))
    return out


# speedup vs baseline: 79.9135x; 1.1880x over previous
import functools

import jax
import jax.numpy as jnp
from jax import lax
from jax.experimental import pallas as pl
from jax.experimental.pallas import tpu as pltpu
from jax.experimental.pallas import tpu_sc as plsc

B, N, K = 16, 2048, 20
TOTAL = B * N
NC, NS = 2, 16
NW = NC * NS
PPW = TOTAL // NW
BF = jnp.bfloat16


def _make_pool(C, fuse_u, G, IDXW):
    ROWS = G * K
    NDMA = ROWS // IDXW
    NCHUNK = PPW // G
    assert ROWS % IDXW == 0 and NCHUNK % 2 == 0
    mesh = plsc.VectorSubcoreMesh(
        core_axis_name="c", subcore_axis_name="s",
        num_cores=NC, num_subcores=NS)
    scratch = [
        [pltpu.VMEM((NDMA, IDXW), jnp.int32) for _ in range(2)],
        [pltpu.VMEM((ROWS, C), BF) for _ in range(2)],
        pltpu.VMEM((G, C), BF),
    ]
    if fuse_u:
        scratch.append([pltpu.VMEM((G, C), BF) for _ in range(2)])
    scratch.append([pltpu.SemaphoreType.DMA for _ in range(2)])

    def body(table_hbm, gidx_hbm, *rest):
        if fuse_u:
            u_hbm, out_hbm, idx_b, rows_b, out_v, u_b, sem_b = rest
        else:
            out_hbm, idx_b, rows_b, out_v, sem_b = rest
        wid = lax.axis_index("s") * NC + lax.axis_index("c")
        base = wid * PPW

        def fire(c, s):
            pltpu.sync_copy(gidx_hbm.at[wid * NCHUNK + c], idx_b[s])
            for j in range(NDMA):
                pltpu.async_copy(table_hbm.at[idx_b[s].at[j]],
                                 rows_b[s].at[pl.ds(j * IDXW, IDXW)],
                                 sem_b[s])
            if fuse_u:
                pltpu.sync_copy(u_hbm.at[pl.ds(base + c * G, G)], u_b[s])

        def drain_compute(c, s):
            for j in range(NDMA):
                pltpu.make_async_copy(
                    table_hbm.at[idx_b[s].at[j]],
                    rows_b[s].at[pl.ds(j * IDXW, IDXW)],
                    sem_b[s]).wait()
            rows_v = rows_b[s]

            def point(g, c2):
                r0 = g * K
                for co in range(C // 32):
                    sl = pl.ds(co * 32, 32)
                    acc = rows_v[r0, sl]
                    for kk in range(1, K):
                        acc = jnp.maximum(acc, rows_v[r0 + kk, sl])
                    if fuse_u:
                        t = acc + u_b[s][g, sl]
                        acc = jnp.maximum(t, BF(0.2) * t)
                    out_v[g, sl] = acc
                return c2

            lax.fori_loop(0, G, point, 0)
            pltpu.sync_copy(out_v, out_hbm.at[pl.ds(base + c * G, G)])

        fire(0, 0)

        def pair(t, carry):
            i0 = 2 * t
            fire(i0 + 1, 1)
            drain_compute(i0, 0)

            @pl.when(i0 + 2 < NCHUNK)
            def _():
                fire(i0 + 2, 0)

            drain_compute(i0 + 1, 1)
            return carry

        lax.fori_loop(0, NCHUNK // 2, pair, 0)

    return pl.kernel(
        body,
        out_type=jax.ShapeDtypeStruct((TOTAL, C), BF),
        mesh=mesh,
        scratch_types=scratch,
        compiler_params=pltpu.CompilerParams(use_tc_tiling_on_sc=False),
    )


def _pu_kernel(x_ref, wp_ref, wu_ref, bu_ref, p_ref, u_ref):
    xb = x_ref[...]
    p = (xb[:, 0:1] * wp_ref[0:1, :] + xb[:, 1:2] * wp_ref[1:2, :]
         + xb[:, 2:3] * wp_ref[2:3, :])
    u = (xb[:, 0:1] * wu_ref[0:1, :] + xb[:, 1:2] * wu_ref[1:2, :]
         + xb[:, 2:3] * wu_ref[2:3, :]) + bu_ref[...]
    p_ref[...] = p.astype(BF)
    u_ref[...] = u.astype(BF)


def _mm_kernel(x_ref, wt_ref, b_ref, o_ref):
    z = jnp.dot(x_ref[...], wt_ref[...],
                preferred_element_type=jnp.float32) + b_ref[...]
    o_ref[...] = jnp.maximum(z, 0.2 * z).astype(BF)


def _final_kernel(h0_ref, h1_ref, h2_ref, h3_ref, wt_ref, b_ref, o_ref):
    cat = jnp.concatenate(
        [h0_ref[...], h1_ref[...], h2_ref[...], h3_ref[...]], axis=1)
    z = jnp.dot(cat, wt_ref[...],
                preferred_element_type=jnp.float32) + b_ref[...]
    z = jnp.maximum(z, 0.2 * z)
    b = pl.program_id(0)
    o_ref[pl.ds(b, 1), :] = jnp.max(z, axis=0, keepdims=True)


def _fold(W, b, g, beta):
    return g[:, None] * W, g * b + beta


def kernel(x, indices, W_edge, b_edge, g_edge, beta_edge,
           W1, b1, g1, beta1, W2, b2, g2, beta2, W3, b3, g3, beta3, Wf, bf):
    f32 = jnp.float32
    We, be = _fold(W_edge, b_edge, g_edge, beta_edge)
    Wc, Wd = We[:, :3], We[:, 3:]
    W1f, b1f = _fold(W1, b1, g1, beta1)
    W2f, b2f = _fold(W2, b2, g2, beta2)
    W3f, b3f = _fold(W3, b3, g3, beta3)

    xf = x.reshape(TOTAL, 3).astype(f32)
    gidx = (indices.astype(jnp.int32)
            + (jnp.arange(B, dtype=jnp.int32) * N)[:, None, None])
    gidx64 = gidx.reshape(NW * (PPW // 64), 10, 128)
    gidx128 = gidx.reshape(NW * (PPW // 32), 5, 128)

    R = 2048
    grid = (TOTAL // R,)
    p, u = pl.pallas_call(
        _pu_kernel,
        grid=grid,
        in_specs=[
            pl.BlockSpec((R, 3), lambda i: (i, 0)),
            pl.BlockSpec((3, 64), lambda i: (0, 0)),
            pl.BlockSpec((3, 64), lambda i: (0, 0)),
            pl.BlockSpec((1, 64), lambda i: (0, 0)),
        ],
        out_specs=[pl.BlockSpec((R, 64), lambda i: (i, 0)),
                   pl.BlockSpec((R, 64), lambda i: (i, 0))],
        out_shape=[jax.ShapeDtypeStruct((TOTAL, 64), BF),
                   jax.ShapeDtypeStruct((TOTAL, 64), BF)],
    )(xf, Wd.T, (Wc - Wd).T, be.reshape(1, 64))

    h0 = _make_pool(64, True, 64, 128)(p, gidx64, u)

    def mm(h, Wl, bl, Cin, Cout):
        return pl.pallas_call(
            _mm_kernel,
            grid=grid,
            in_specs=[
                pl.BlockSpec((R, Cin), lambda i: (i, 0)),
                pl.BlockSpec((Cin, Cout), lambda i: (0, 0)),
                pl.BlockSpec((1, Cout), lambda i: (0, 0)),
            ],
            out_specs=pl.BlockSpec((R, Cout), lambda i: (i, 0)),
            out_shape=jax.ShapeDtypeStruct((TOTAL, Cout), BF),
        )(h, Wl.T.astype(BF), bl.reshape(1, Cout))

    pool64 = _make_pool(64, False, 64, 128)
    h1 = mm(pool64(h0, gidx64), W1f, b1f, 64, 64)
    h2 = mm(pool64(h1, gidx64), W2f, b2f, 64, 128)
    h3 = mm(_make_pool(128, False, 32, 128)(h2, gidx128), W3f, b3f, 128, 256)

    out = pl.pallas_call(
        _final_kernel,
        grid=(B,),
        in_specs=[
            pl.BlockSpec((N, 64), lambda b: (b, 0)),
            pl.BlockSpec((N, 64), lambda b: (b, 0)),
            pl.BlockSpec((N, 128), lambda b: (b, 0)),
            pl.BlockSpec((N, 256), lambda b: (b, 0)),
            pl.BlockSpec((512, 512), lambda b: (0, 0)),
            pl.BlockSpec((1, 512), lambda b: (0, 0)),
        ],
        out_specs=pl.BlockSpec((B, 512), lambda b: (0, 0)),
        out_shape=jax.ShapeDtypeStruct((B, 512), f32),
    )(h0, h1, h2, h3, Wf.T.astype(BF), bf.reshape(1, 512))
    return out


# speedup vs baseline: 98.9376x; 1.2381x over previous
import functools

import jax
import jax.numpy as jnp
from jax import lax
from jax.experimental import pallas as pl
from jax.experimental.pallas import tpu as pltpu
from jax.experimental.pallas import tpu_sc as plsc

B, N, K = 16, 2048, 20
BH = B // 2
TOT = BH * N
NC, NS = 2, 16
NW = NC * NS
PPW = TOT // NW
BF = jnp.bfloat16


def _make_pool(C, fuse_u, G, IDXW):
    ROWS = G * K
    NDMA = ROWS // IDXW
    NCHUNK = PPW // G
    assert ROWS % IDXW == 0 and NCHUNK % 2 == 0
    mesh = plsc.VectorSubcoreMesh(
        core_axis_name="c", subcore_axis_name="s",
        num_cores=NC, num_subcores=NS)
    scratch = [
        [pltpu.VMEM((NDMA, IDXW), jnp.int32) for _ in range(2)],
        [pltpu.VMEM((ROWS, C), BF) for _ in range(2)],
        pltpu.VMEM((G, C), BF),
    ]
    if fuse_u:
        scratch.append([pltpu.VMEM((G, C), BF) for _ in range(2)])
    scratch.append([pltpu.SemaphoreType.DMA for _ in range(2)])

    def body(table_hbm, gidx_hbm, *rest):
        if fuse_u:
            u_hbm, out_hbm, idx_b, rows_b, out_v, u_b, sem_b = rest
        else:
            out_hbm, idx_b, rows_b, out_v, sem_b = rest
        wid = lax.axis_index("s") * NC + lax.axis_index("c")
        base = wid * PPW

        def fire(c, s):
            pltpu.sync_copy(gidx_hbm.at[wid * NCHUNK + c], idx_b[s])
            for j in range(NDMA):
                pltpu.async_copy(table_hbm.at[idx_b[s].at[j]],
                                 rows_b[s].at[pl.ds(j * IDXW, IDXW)],
                                 sem_b[s])
            if fuse_u:
                pltpu.sync_copy(u_hbm.at[pl.ds(base + c * G, G)], u_b[s])

        def drain_compute(c, s):
            for j in range(NDMA):
                pltpu.make_async_copy(
                    table_hbm.at[idx_b[s].at[j]],
                    rows_b[s].at[pl.ds(j * IDXW, IDXW)],
                    sem_b[s]).wait()
            rows_v = rows_b[s]

            def point(g, c2):
                r0 = g * K
                for co in range(C // 32):
                    sl = pl.ds(co * 32, 32)
                    acc = rows_v[r0, sl]
                    for kk in range(1, K):
                        acc = jnp.maximum(acc, rows_v[r0 + kk, sl])
                    if fuse_u:
                        t = acc + u_b[s][g, sl]
                        acc = jnp.maximum(t, BF(0.2) * t)
                    out_v[g, sl] = acc
                return c2

            lax.fori_loop(0, G, point, 0)
            pltpu.sync_copy(out_v, out_hbm.at[pl.ds(base + c * G, G)])

        fire(0, 0)

        def pair(t, carry):
            i0 = 2 * t
            fire(i0 + 1, 1)
            drain_compute(i0, 0)

            @pl.when(i0 + 2 < NCHUNK)
            def _():
                fire(i0 + 2, 0)

            drain_compute(i0 + 1, 1)
            return carry

        lax.fori_loop(0, NCHUNK // 2, pair, 0)

    return pl.kernel(
        body,
        out_type=jax.ShapeDtypeStruct((TOT, C), BF),
        mesh=mesh,
        scratch_types=scratch,
        compiler_params=pltpu.CompilerParams(use_tc_tiling_on_sc=False),
    )


def _pu_kernel(x_ref, wp_ref, wu_ref, bu_ref, p_ref, u_ref):
    xb = x_ref[...]
    p = (xb[:, 0:1] * wp_ref[0:1, :] + xb[:, 1:2] * wp_ref[1:2, :]
         + xb[:, 2:3] * wp_ref[2:3, :])
    u = (xb[:, 0:1] * wu_ref[0:1, :] + xb[:, 1:2] * wu_ref[1:2, :]
         + xb[:, 2:3] * wu_ref[2:3, :]) + bu_ref[...]
    p_ref[...] = p.astype(BF)
    u_ref[...] = u.astype(BF)


def _mm_kernel(x_ref, wt_ref, b_ref, o_ref):
    z = jnp.dot(x_ref[...], wt_ref[...],
                preferred_element_type=jnp.float32) + b_ref[...]
    o_ref[...] = jnp.maximum(z, 0.2 * z).astype(BF)


def _final_kernel(h0_ref, h1_ref, h2_ref, h3_ref, wt_ref, b_ref, o_ref):
    cat = jnp.concatenate(
        [h0_ref[...], h1_ref[...], h2_ref[...], h3_ref[...]], axis=1)
    z = jnp.dot(cat, wt_ref[...],
                preferred_element_type=jnp.float32) + b_ref[...]
    z = jnp.maximum(z, 0.2 * z)
    b = pl.program_id(0)
    o_ref[pl.ds(b, 1), :] = jnp.max(z, axis=0, keepdims=True)


def _fold(W, b, g, beta):
    return g[:, None] * W, g * b + beta


_R = 2048


def _half(xh, gidx64, gidx128, WpT, WuT, beT, W1t, b1r, W2t, b2r,
          W3t, b3r, WfT, bfr):
    f32 = jnp.float32
    grid = (TOT // _R,)
    p, u = pl.pallas_call(
        _pu_kernel,
        grid=grid,
        in_specs=[
            pl.BlockSpec((_R, 3), lambda i: (i, 0)),
            pl.BlockSpec((3, 64), lambda i: (0, 0)),
            pl.BlockSpec((3, 64), lambda i: (0, 0)),
            pl.BlockSpec((1, 64), lambda i: (0, 0)),
        ],
        out_specs=[pl.BlockSpec((_R, 64), lambda i: (i, 0)),
                   pl.BlockSpec((_R, 64), lambda i: (i, 0))],
        out_shape=[jax.ShapeDtypeStruct((TOT, 64), BF),
                   jax.ShapeDtypeStruct((TOT, 64), BF)],
    )(xh, WpT, WuT, beT)

    h0 = _make_pool(64, True, 64, 128)(p, gidx64, u)

    def mm(h, Wt, br, Cin, Cout):
        return pl.pallas_call(
            _mm_kernel,
            grid=grid,
            in_specs=[
                pl.BlockSpec((_R, Cin), lambda i: (i, 0)),
                pl.BlockSpec((Cin, Cout), lambda i: (0, 0)),
                pl.BlockSpec((1, Cout), lambda i: (0, 0)),
            ],
            out_specs=pl.BlockSpec((_R, Cout), lambda i: (i, 0)),
            out_shape=jax.ShapeDtypeStruct((TOT, Cout), BF),
        )(h, Wt, br)

    pool64 = _make_pool(64, False, 64, 128)
    h1 = mm(pool64(h0, gidx64), W1t, b1r, 64, 64)
    h2 = mm(pool64(h1, gidx64), W2t, b2r, 64, 128)
    h3 = mm(_make_pool(128, False, 32, 128)(h2, gidx128), W3t, b3r, 128, 256)

    return pl.pallas_call(
        _final_kernel,
        grid=(BH,),
        in_specs=[
            pl.BlockSpec((N, 64), lambda b: (b, 0)),
            pl.BlockSpec((N, 64), lambda b: (b, 0)),
            pl.BlockSpec((N, 128), lambda b: (b, 0)),
            pl.BlockSpec((N, 256), lambda b: (b, 0)),
            pl.BlockSpec((512, 512), lambda b: (0, 0)),
            pl.BlockSpec((1, 512), lambda b: (0, 0)),
        ],
        out_specs=pl.BlockSpec((BH, 512), lambda b: (0, 0)),
        out_shape=jax.ShapeDtypeStruct((BH, 512), f32),
    )(h0, h1, h2, h3, WfT, bfr)


def kernel(x, indices, W_edge, b_edge, g_edge, beta_edge,
           W1, b1, g1, beta1, W2, b2, g2, beta2, W3, b3, g3, beta3, Wf, bf):
    f32 = jnp.float32
    We, be = _fold(W_edge, b_edge, g_edge, beta_edge)
    Wc, Wd = We[:, :3], We[:, 3:]
    W1f, b1f = _fold(W1, b1, g1, beta1)
    W2f, b2f = _fold(W2, b2, g2, beta2)
    W3f, b3f = _fold(W3, b3, g3, beta3)

    xf = x.reshape(B * N, 3).astype(f32)
    idx32 = indices.astype(jnp.int32)
    off = (jnp.arange(BH, dtype=jnp.int32) * N)[:, None, None]
    halves = []
    wargs = (Wd.T, (Wc - Wd).T, be.reshape(1, 64),
             W1f.T.astype(BF), b1f.reshape(1, 64),
             W2f.T.astype(BF), b2f.reshape(1, 128),
             W3f.T.astype(BF), b3f.reshape(1, 256),
             Wf.T.astype(BF), bf.reshape(1, 512))
    for h in range(2):
        gidx = idx32[h * BH:(h + 1) * BH] + off
        gidx64 = gidx.reshape(NW * (PPW // 64), 10, 128)
        gidx128 = gidx.reshape(NW * (PPW // 32), 5, 128)
        xh = xf[h * TOT:(h + 1) * TOT]
        halves.append(_half(xh, gidx64, gidx128, *wargs))
    return jnp.concatenate(halves, axis=0)


# speedup vs baseline: 99.6827x; 1.0075x over previous
import functools

import jax
import jax.numpy as jnp
from jax import lax
from jax.experimental import pallas as pl
from jax.experimental.pallas import tpu as pltpu
from jax.experimental.pallas import tpu_sc as plsc

B, N, K = 16, 2048, 20
BH = B // 2
TOT = BH * N
NC, NS = 2, 16
NW = NC * NS
PPW = TOT // NW
BF = jnp.bfloat16


def _make_pool(C, fuse_u, G, IDXW):
    ROWS = G * K
    NDMA = ROWS // IDXW
    NCHUNK = PPW // G
    assert ROWS % IDXW == 0 and NCHUNK % 2 == 0
    mesh = plsc.VectorSubcoreMesh(
        core_axis_name="c", subcore_axis_name="s",
        num_cores=NC, num_subcores=NS)
    scratch = [
        [pltpu.VMEM((NDMA, IDXW), jnp.int32) for _ in range(2)],
        [pltpu.VMEM((ROWS, C), BF) for _ in range(2)],
        pltpu.VMEM((G, C), BF),
    ]
    if fuse_u:
        scratch.append([pltpu.VMEM((G, C), BF) for _ in range(2)])
    scratch.append([pltpu.SemaphoreType.DMA for _ in range(2)])

    def body(table_hbm, gidx_hbm, *rest):
        if fuse_u:
            u_hbm, out_hbm, idx_b, rows_b, out_v, u_b, sem_b = rest
        else:
            out_hbm, idx_b, rows_b, out_v, sem_b = rest
        wid = lax.axis_index("s") * NC + lax.axis_index("c")
        base = wid * PPW

        def fire(c, s):
            pltpu.sync_copy(gidx_hbm.at[wid * NCHUNK + c], idx_b[s])
            for j in range(NDMA):
                pltpu.async_copy(table_hbm.at[idx_b[s].at[j]],
                                 rows_b[s].at[pl.ds(j * IDXW, IDXW)],
                                 sem_b[s])
            if fuse_u:
                pltpu.sync_copy(u_hbm.at[pl.ds(base + c * G, G)], u_b[s])

        def drain_compute(c, s):
            for j in range(NDMA):
                pltpu.make_async_copy(
                    table_hbm.at[idx_b[s].at[j]],
                    rows_b[s].at[pl.ds(j * IDXW, IDXW)],
                    sem_b[s]).wait()
            rows_v = rows_b[s]

            def point(g, c2):
                r0 = g * K
                for co in range(C // 32):
                    sl = pl.ds(co * 32, 32)
                    acc = rows_v[r0, sl]
                    for kk in range(1, K):
                        acc = jnp.maximum(acc, rows_v[r0 + kk, sl])
                    if fuse_u:
                        t = acc + u_b[s][g, sl]
                        acc = jnp.maximum(t, BF(0.2) * t)
                    out_v[g, sl] = acc
                return c2

            lax.fori_loop(0, G, point, 0)
            pltpu.sync_copy(out_v, out_hbm.at[pl.ds(base + c * G, G)])

        fire(0, 0)

        def pair(t, carry):
            i0 = 2 * t
            fire(i0 + 1, 1)
            drain_compute(i0, 0)

            @pl.when(i0 + 2 < NCHUNK)
            def _():
                fire(i0 + 2, 0)

            drain_compute(i0 + 1, 1)
            return carry

        lax.fori_loop(0, NCHUNK // 2, pair, 0)

    return pl.kernel(
        body,
        out_type=jax.ShapeDtypeStruct((TOT, C), BF),
        mesh=mesh,
        scratch_types=scratch,
        compiler_params=pltpu.CompilerParams(use_tc_tiling_on_sc=False),
    )


def _make_pu():
    mesh = plsc.VectorSubcoreMesh(
        core_axis_name="c", subcore_axis_name="s",
        num_cores=NC, num_subcores=NS)
    scratch = [
        pltpu.VMEM((PPW * 3 + 16,), jnp.float32),
        pltpu.VMEM((3, 64), jnp.float32),
        pltpu.VMEM((3, 64), jnp.float32),
        pltpu.VMEM((1, 64), jnp.float32),
        pltpu.VMEM((PPW, 64), BF),
        pltpu.VMEM((PPW, 64), BF),
    ]

    def body(x_hbm, wp_hbm, wu_hbm, bu_hbm, p_hbm, u_hbm,
             x_v, wp_v, wu_v, bu_v, p_v, u_v):
        wid = lax.axis_index("s") * NC + lax.axis_index("c")
        base = wid * PPW
        pltpu.sync_copy(x_hbm.at[pl.ds(base * 3, PPW * 3)],
                        x_v.at[pl.ds(0, PPW * 3)])
        pltpu.sync_copy(wp_hbm, wp_v)
        pltpu.sync_copy(wu_hbm, wu_v)
        pltpu.sync_copy(bu_hbm, bu_v)

        def point(g, c2):
            xv = x_v[pl.ds(3 * g, 16)]
            x0, x1, x2 = xv[0], xv[1], xv[2]
            for blk in range(2):
                sls = [pl.ds(blk * 32 + h * 16, 16) for h in range(2)]
                pg = [x0 * wp_v[0, sl] + x1 * wp_v[1, sl] + x2 * wp_v[2, sl]
                      for sl in sls]
                p_v[g, pl.ds(blk * 32, 32)] = plsc.pack(pg[0], pg[1], format=plsc.PackFormat.INTERLEAVED)
                ug = [x0 * wu_v[0, sl] + x1 * wu_v[1, sl] + x2 * wu_v[2, sl]
                      + bu_v[0, sl] for sl in sls]
                u_v[g, pl.ds(blk * 32, 32)] = plsc.pack(ug[0], ug[1], format=plsc.PackFormat.INTERLEAVED)
            return c2

        lax.fori_loop(0, PPW, point, 0)
        pltpu.sync_copy(p_v, p_hbm.at[pl.ds(base, PPW)])
        pltpu.sync_copy(u_v, u_hbm.at[pl.ds(base, PPW)])

    return pl.kernel(
        body,
        out_type=[jax.ShapeDtypeStruct((TOT, 64), BF),
                  jax.ShapeDtypeStruct((TOT, 64), BF)],
        mesh=mesh,
        scratch_types=scratch,
        compiler_params=pltpu.CompilerParams(
            use_tc_tiling_on_sc=False, needs_layout_passes=False),
    )


def _mm_kernel(x_ref, wt_ref, b_ref, o_ref):
    z = jnp.dot(x_ref[...], wt_ref[...],
                preferred_element_type=jnp.float32) + b_ref[...]
    o_ref[...] = jnp.maximum(z, 0.2 * z).astype(BF)


def _final_kernel(h0_ref, h1_ref, h2_ref, h3_ref, wt_ref, b_ref, o_ref):
    cat = jnp.concatenate(
        [h0_ref[...], h1_ref[...], h2_ref[...], h3_ref[...]], axis=1)
    z = jnp.dot(cat, wt_ref[...],
                preferred_element_type=jnp.float32) + b_ref[...]
    z = jnp.maximum(z, 0.2 * z)
    b = pl.program_id(0)
    o_ref[pl.ds(b, 1), :] = jnp.max(z, axis=0, keepdims=True)


def _fold(W, b, g, beta):
    return g[:, None] * W, g * b + beta


_R = 2048


def _half(xh, gidx64, gidx128, WpT, WuT, beT, W1t, b1r, W2t, b2r,
          W3t, b3r, WfT, bfr):
    f32 = jnp.float32
    grid = (TOT // _R,)
    p, u = _make_pu()(xh, WpT, WuT, beT)

    h0 = _make_pool(64, True, 64, 128)(p, gidx64, u)

    def mm(h, Wt, br, Cin, Cout):
        return pl.pallas_call(
            _mm_kernel,
            grid=grid,
            in_specs=[
                pl.BlockSpec((_R, Cin), lambda i: (i, 0)),
                pl.BlockSpec((Cin, Cout), lambda i: (0, 0)),
                pl.BlockSpec((1, Cout), lambda i: (0, 0)),
            ],
            out_specs=pl.BlockSpec((_R, Cout), lambda i: (i, 0)),
            out_shape=jax.ShapeDtypeStruct((TOT, Cout), BF),
        )(h, Wt, br)

    pool64 = _make_pool(64, False, 64, 128)
    h1 = mm(pool64(h0, gidx64), W1t, b1r, 64, 64)
    h2 = mm(pool64(h1, gidx64), W2t, b2r, 64, 128)
    h3 = mm(_make_pool(128, False, 32, 128)(h2, gidx128), W3t, b3r, 128, 256)

    return pl.pallas_call(
        _final_kernel,
        grid=(BH,),
        in_specs=[
            pl.BlockSpec((N, 64), lambda b: (b, 0)),
            pl.BlockSpec((N, 64), lambda b: (b, 0)),
            pl.BlockSpec((N, 128), lambda b: (b, 0)),
            pl.BlockSpec((N, 256), lambda b: (b, 0)),
            pl.BlockSpec((512, 512), lambda b: (0, 0)),
            pl.BlockSpec((1, 512), lambda b: (0, 0)),
        ],
        out_specs=pl.BlockSpec((BH, 512), lambda b: (0, 0)),
        out_shape=jax.ShapeDtypeStruct((BH, 512), f32),
    )(h0, h1, h2, h3, WfT, bfr)


def kernel(x, indices, W_edge, b_edge, g_edge, beta_edge,
           W1, b1, g1, beta1, W2, b2, g2, beta2, W3, b3, g3, beta3, Wf, bf):
    f32 = jnp.float32
    We, be = _fold(W_edge, b_edge, g_edge, beta_edge)
    Wc, Wd = We[:, :3], We[:, 3:]
    W1f, b1f = _fold(W1, b1, g1, beta1)
    W2f, b2f = _fold(W2, b2, g2, beta2)
    W3f, b3f = _fold(W3, b3, g3, beta3)

    xf = x.reshape(B * N, 3).astype(f32)
    idx32 = indices.astype(jnp.int32)
    off = (jnp.arange(BH, dtype=jnp.int32) * N)[:, None, None]
    import numpy as _np
    perm = _np.concatenate(
        [_np.concatenate([_np.arange(0, 32, 2), _np.arange(1, 32, 2)])
         + 32 * blk for blk in range(2)])
    halves = []
    wargs = (Wd.T[:, perm], (Wc - Wd).T[:, perm], be[perm].reshape(1, 64),
             W1f.T.astype(BF), b1f.reshape(1, 64),
             W2f.T.astype(BF), b2f.reshape(1, 128),
             W3f.T.astype(BF), b3f.reshape(1, 256),
             Wf.T.astype(BF), bf.reshape(1, 512))
    for h in range(2):
        gidx = idx32[h * BH:(h + 1) * BH] + off
        gidx64 = gidx.reshape(NW * (PPW // 64), 10, 128)
        gidx128 = gidx.reshape(NW * (PPW // 32), 5, 128)
        xh = xf.reshape(B * N * 3)[h * TOT * 3:(h + 1) * TOT * 3]
        halves.append(_half(xh, gidx64, gidx128, *wargs))
    return jnp.concatenate(halves, axis=0)


# speedup vs baseline: 103.2575x; 1.0359x over previous
import functools

import jax
import jax.numpy as jnp
from jax import lax
from jax.experimental import pallas as pl
from jax.experimental.pallas import tpu as pltpu
from jax.experimental.pallas import tpu_sc as plsc

B, N, K = 16, 2048, 20
BH = B // 2
TOT = BH * N
NC, NS = 2, 16
NW = NC * NS
PPW = TOT // NW
BF = jnp.bfloat16


def _make_pool(C, fuse_u, G, IDXW):
    ROWS = G * K
    NDMA = ROWS // IDXW
    NCHUNK = PPW // G
    assert ROWS % IDXW == 0 and NCHUNK % 2 == 0
    mesh = plsc.VectorSubcoreMesh(
        core_axis_name="c", subcore_axis_name="s",
        num_cores=NC, num_subcores=NS)
    scratch = [
        [pltpu.VMEM((NDMA, IDXW), jnp.int32) for _ in range(2)],
        [pltpu.VMEM((ROWS, C), BF) for _ in range(2)],
        pltpu.VMEM((G, C), BF),
    ]
    if fuse_u:
        scratch.append([pltpu.VMEM((G, C), BF) for _ in range(2)])
    scratch.append([pltpu.SemaphoreType.DMA for _ in range(2)])

    def body(table_hbm, gidx_hbm, *rest):
        if fuse_u:
            u_hbm, out_hbm, idx_b, rows_b, out_v, u_b, sem_b = rest
        else:
            out_hbm, idx_b, rows_b, out_v, sem_b = rest
        wid = lax.axis_index("s") * NC + lax.axis_index("c")
        base = wid * PPW

        def fire(c, s):
            pltpu.sync_copy(gidx_hbm.at[wid * NCHUNK + c], idx_b[s])
            for j in range(NDMA):
                pltpu.async_copy(table_hbm.at[idx_b[s].at[j]],
                                 rows_b[s].at[pl.ds(j * IDXW, IDXW)],
                                 sem_b[s])
            if fuse_u:
                pltpu.sync_copy(u_hbm.at[pl.ds(base + c * G, G)], u_b[s])

        def drain_compute(c, s):
            for j in range(NDMA):
                pltpu.make_async_copy(
                    table_hbm.at[idx_b[s].at[j]],
                    rows_b[s].at[pl.ds(j * IDXW, IDXW)],
                    sem_b[s]).wait()
            rows_v = rows_b[s]

            def point(g, c2):
                r0 = g * K
                for co in range(C // 32):
                    sl = pl.ds(co * 32, 32)
                    acc = rows_v[r0, sl]
                    for kk in range(1, K):
                        acc = jnp.maximum(acc, rows_v[r0 + kk, sl])
                    if fuse_u:
                        t = acc + u_b[s][g, sl]
                        acc = jnp.maximum(t, BF(0.2) * t)
                    out_v[g, sl] = acc
                return c2

            lax.fori_loop(0, G, point, 0)
            pltpu.sync_copy(out_v, out_hbm.at[pl.ds(base + c * G, G)])

        fire(0, 0)

        def pair(t, carry):
            i0 = 2 * t
            fire(i0 + 1, 1)
            drain_compute(i0, 0)

            @pl.when(i0 + 2 < NCHUNK)
            def _():
                fire(i0 + 2, 0)

            drain_compute(i0 + 1, 1)
            return carry

        lax.fori_loop(0, NCHUNK // 2, pair, 0)

    return pl.kernel(
        body,
        out_type=jax.ShapeDtypeStruct((TOT, C), BF),
        mesh=mesh,
        scratch_types=scratch,
        compiler_params=pltpu.CompilerParams(use_tc_tiling_on_sc=False),
    )


def _make_pu():
    mesh = plsc.VectorSubcoreMesh(
        core_axis_name="c", subcore_axis_name="s",
        num_cores=NC, num_subcores=NS)
    scratch = [
        pltpu.VMEM((PPW * 3 + 16,), jnp.float32),
        pltpu.VMEM((3, 64), jnp.float32),
        pltpu.VMEM((3, 64), jnp.float32),
        pltpu.VMEM((1, 64), jnp.float32),
        pltpu.VMEM((PPW, 64), BF),
        pltpu.VMEM((PPW, 64), BF),
    ]

    def body(x_hbm, wp_hbm, wu_hbm, bu_hbm, p_hbm, u_hbm,
             x_v, wp_v, wu_v, bu_v, p_v, u_v):
        wid = lax.axis_index("s") * NC + lax.axis_index("c")
        base = wid * PPW
        pltpu.sync_copy(x_hbm.at[pl.ds(base * 3, PPW * 3)],
                        x_v.at[pl.ds(0, PPW * 3)])
        pltpu.sync_copy(wp_hbm, wp_v)
        pltpu.sync_copy(wu_hbm, wu_v)
        pltpu.sync_copy(bu_hbm, bu_v)

        sls = [pl.ds(q * 16, 16) for q in range(4)]
        wp = [[wp_v[c, sl] for sl in sls] for c in range(3)]
        wu = [[wu_v[c, sl] for sl in sls] for c in range(3)]
        bu = [bu_v[0, sl] for sl in sls]

        def point(g, c2):
            xv = x_v[pl.ds(3 * g, 16)]
            x0, x1, x2 = xv[0], xv[1], xv[2]
            for blk in range(2):
                q0, q1 = 2 * blk, 2 * blk + 1
                pg = [x0 * wp[0][q] + x1 * wp[1][q] + x2 * wp[2][q]
                      for q in (q0, q1)]
                p_v[g, pl.ds(blk * 32, 32)] = plsc.pack(
                    pg[0], pg[1], format=plsc.PackFormat.INTERLEAVED)
                ug = [x0 * wu[0][q] + x1 * wu[1][q] + x2 * wu[2][q] + bu[q]
                      for q in (q0, q1)]
                u_v[g, pl.ds(blk * 32, 32)] = plsc.pack(
                    ug[0], ug[1], format=plsc.PackFormat.INTERLEAVED)
            return c2

        lax.fori_loop(0, PPW, point, 0)
        pltpu.sync_copy(p_v, p_hbm.at[pl.ds(base, PPW)])
        pltpu.sync_copy(u_v, u_hbm.at[pl.ds(base, PPW)])

    return pl.kernel(
        body,
        out_type=[jax.ShapeDtypeStruct((TOT, 64), BF),
                  jax.ShapeDtypeStruct((TOT, 64), BF)],
        mesh=mesh,
        scratch_types=scratch,
        compiler_params=pltpu.CompilerParams(
            use_tc_tiling_on_sc=False, needs_layout_passes=False),
    )


def _mm_kernel(x_ref, wt_ref, b_ref, o_ref):
    z = jnp.dot(x_ref[...], wt_ref[...],
                preferred_element_type=jnp.float32) + b_ref[...]
    o_ref[...] = jnp.maximum(z, 0.2 * z).astype(BF)


def _final_kernel(h0_ref, h1_ref, h2_ref, h3_ref, wt_ref, b_ref, o_ref):
    cat = jnp.concatenate(
        [h0_ref[...], h1_ref[...], h2_ref[...], h3_ref[...]], axis=1)
    z = jnp.dot(cat, wt_ref[...],
                preferred_element_type=jnp.float32) + b_ref[...]
    z = jnp.maximum(z, 0.2 * z)
    b = pl.program_id(0)
    o_ref[pl.ds(b, 1), :] = jnp.max(z, axis=0, keepdims=True)


def _fold(W, b, g, beta):
    return g[:, None] * W, g * b + beta


_R = 2048


def _half(xh, gidx64, gidx128, WpT, WuT, beT, W1t, b1r, W2t, b2r,
          W3t, b3r, WfT, bfr):
    f32 = jnp.float32
    grid = (TOT // _R,)
    p, u = _make_pu()(xh, WpT, WuT, beT)

    h0 = _make_pool(64, True, 64, 128)(p, gidx64, u)

    def mm(h, Wt, br, Cin, Cout):
        return pl.pallas_call(
            _mm_kernel,
            grid=grid,
            in_specs=[
                pl.BlockSpec((_R, Cin), lambda i: (i, 0)),
                pl.BlockSpec((Cin, Cout), lambda i: (0, 0)),
                pl.BlockSpec((1, Cout), lambda i: (0, 0)),
            ],
            out_specs=pl.BlockSpec((_R, Cout), lambda i: (i, 0)),
            out_shape=jax.ShapeDtypeStruct((TOT, Cout), BF),
        )(h, Wt, br)

    pool64 = _make_pool(64, False, 64, 128)
    h1 = mm(pool64(h0, gidx64), W1t, b1r, 64, 64)
    h2 = mm(pool64(h1, gidx64), W2t, b2r, 64, 128)
    h3 = mm(_make_pool(128, False, 32, 128)(h2, gidx128), W3t, b3r, 128, 256)

    return pl.pallas_call(
        _final_kernel,
        grid=(BH,),
        in_specs=[
            pl.BlockSpec((N, 64), lambda b: (b, 0)),
            pl.BlockSpec((N, 64), lambda b: (b, 0)),
            pl.BlockSpec((N, 128), lambda b: (b, 0)),
            pl.BlockSpec((N, 256), lambda b: (b, 0)),
            pl.BlockSpec((512, 512), lambda b: (0, 0)),
            pl.BlockSpec((1, 512), lambda b: (0, 0)),
        ],
        out_specs=pl.BlockSpec((BH, 512), lambda b: (0, 0)),
        out_shape=jax.ShapeDtypeStruct((BH, 512), f32),
    )(h0, h1, h2, h3, WfT, bfr)


def kernel(x, indices, W_edge, b_edge, g_edge, beta_edge,
           W1, b1, g1, beta1, W2, b2, g2, beta2, W3, b3, g3, beta3, Wf, bf):
    f32 = jnp.float32
    We, be = _fold(W_edge, b_edge, g_edge, beta_edge)
    Wc, Wd = We[:, :3], We[:, 3:]
    W1f, b1f = _fold(W1, b1, g1, beta1)
    W2f, b2f = _fold(W2, b2, g2, beta2)
    W3f, b3f = _fold(W3, b3, g3, beta3)

    xf = x.reshape(B * N, 3).astype(f32)
    idx32 = indices.astype(jnp.int32)
    off = (jnp.arange(BH, dtype=jnp.int32) * N)[:, None, None]
    import numpy as _np
    perm = _np.concatenate(
        [_np.concatenate([_np.arange(0, 32, 2), _np.arange(1, 32, 2)])
         + 32 * blk for blk in range(2)])
    halves = []
    wargs = (Wd.T[:, perm], (Wc - Wd).T[:, perm], be[perm].reshape(1, 64),
             W1f.T.astype(BF), b1f.reshape(1, 64),
             W2f.T.astype(BF), b2f.reshape(1, 128),
             W3f.T.astype(BF), b3f.reshape(1, 256),
             Wf.T.astype(BF), bf.reshape(1, 512))
    for h in range(2):
        gidx = idx32[h * BH:(h + 1) * BH] + off
        gidx64 = gidx.reshape(NW * (PPW // 64), 10, 128)
        gidx128 = gidx.reshape(NW * (PPW // 32), 5, 128)
        xh = xf.reshape(B * N * 3)[h * TOT * 3:(h + 1) * TOT * 3]
        halves.append(_half(xh, gidx64, gidx128, *wargs))
    return jnp.concatenate(halves, axis=0)


# speedup vs baseline: 105.3093x; 1.0199x over previous
import functools

import jax
import jax.numpy as jnp
from jax import lax
from jax.experimental import pallas as pl
from jax.experimental.pallas import tpu as pltpu
from jax.experimental.pallas import tpu_sc as plsc

B, N, K = 16, 2048, 20
BH = B // 2
TOT = BH * N
NC, NS = 2, 16
NW = NC * NS
PPW = TOT // NW
BF = jnp.bfloat16


def _make_pool(C, fuse_u, G, IDXW):
    ROWS = G * K
    NDMA = ROWS // IDXW
    NCHUNK = PPW // G
    assert ROWS % IDXW == 0 and NCHUNK % 2 == 0
    mesh = plsc.VectorSubcoreMesh(
        core_axis_name="c", subcore_axis_name="s",
        num_cores=NC, num_subcores=NS)
    scratch = [
        [pltpu.VMEM((NDMA, IDXW), jnp.int32) for _ in range(2)],
        [pltpu.VMEM((ROWS, C), BF) for _ in range(2)],
        pltpu.VMEM((G, C), BF),
    ]
    if fuse_u:
        scratch.append([pltpu.VMEM((G, C), BF) for _ in range(2)])
        scratch.append([pltpu.SemaphoreType.DMA for _ in range(2)])
    scratch.append([pltpu.SemaphoreType.DMA for _ in range(2)])

    def body(table_hbm, gidx_hbm, *rest):
        if fuse_u:
            u_hbm, out_hbm, idx_b, rows_b, out_v, u_b, usem_b, sem_b = rest
        else:
            out_hbm, idx_b, rows_b, out_v, sem_b = rest
        wid = lax.axis_index("s") * NC + lax.axis_index("c")
        base = wid * PPW

        def fire(c, s):
            pltpu.sync_copy(gidx_hbm.at[wid * NCHUNK + c], idx_b[s])
            for j in range(NDMA):
                pltpu.async_copy(table_hbm.at[idx_b[s].at[j]],
                                 rows_b[s].at[pl.ds(j * IDXW, IDXW)],
                                 sem_b[s])
            if fuse_u:
                pltpu.async_copy(u_hbm.at[pl.ds(base + c * G, G)], u_b[s],
                                 usem_b[s])

        def drain_compute(c, s):
            for j in range(NDMA):
                pltpu.make_async_copy(
                    table_hbm.at[idx_b[s].at[j]],
                    rows_b[s].at[pl.ds(j * IDXW, IDXW)],
                    sem_b[s]).wait()
            if fuse_u:
                pltpu.make_async_copy(u_hbm.at[pl.ds(base + c * G, G)],
                                      u_b[s], usem_b[s]).wait()
            rows_v = rows_b[s]

            def point(g, c2):
                r0 = g * K
                for co in range(C // 32):
                    sl = pl.ds(co * 32, 32)
                    acc = rows_v[r0, sl]
                    for kk in range(1, K):
                        acc = jnp.maximum(acc, rows_v[r0 + kk, sl])
                    if fuse_u:
                        t = acc + u_b[s][g, sl]
                        acc = jnp.maximum(t, BF(0.2) * t)
                    out_v[g, sl] = acc
                return c2

            lax.fori_loop(0, G, point, 0)
            pltpu.sync_copy(out_v, out_hbm.at[pl.ds(base + c * G, G)])

        fire(0, 0)

        def pair(t, carry):
            i0 = 2 * t
            fire(i0 + 1, 1)
            drain_compute(i0, 0)

            @pl.when(i0 + 2 < NCHUNK)
            def _():
                fire(i0 + 2, 0)

            drain_compute(i0 + 1, 1)
            return carry

        lax.fori_loop(0, NCHUNK // 2, pair, 0)

    return pl.kernel(
        body,
        out_type=jax.ShapeDtypeStruct((TOT, C), BF),
        mesh=mesh,
        scratch_types=scratch,
        compiler_params=pltpu.CompilerParams(use_tc_tiling_on_sc=False),
    )


def _make_pu():
    mesh = plsc.VectorSubcoreMesh(
        core_axis_name="c", subcore_axis_name="s",
        num_cores=NC, num_subcores=NS)
    scratch = [
        pltpu.VMEM((PPW * 3 + 16,), jnp.float32),
        pltpu.VMEM((3, 64), jnp.float32),
        pltpu.VMEM((3, 64), jnp.float32),
        pltpu.VMEM((1, 64), jnp.float32),
        pltpu.VMEM((PPW, 64), BF),
        pltpu.VMEM((PPW, 64), BF),
    ]

    def body(x_hbm, wp_hbm, wu_hbm, bu_hbm, p_hbm, u_hbm,
             x_v, wp_v, wu_v, bu_v, p_v, u_v):
        wid = lax.axis_index("s") * NC + lax.axis_index("c")
        base = wid * PPW
        pltpu.sync_copy(x_hbm.at[pl.ds(base * 3, PPW * 3)],
                        x_v.at[pl.ds(0, PPW * 3)])
        pltpu.sync_copy(wp_hbm, wp_v)
        pltpu.sync_copy(wu_hbm, wu_v)
        pltpu.sync_copy(bu_hbm, bu_v)

        sls = [pl.ds(q * 16, 16) for q in range(4)]
        wp = [[wp_v[c, sl] for sl in sls] for c in range(3)]
        wu = [[wu_v[c, sl] for sl in sls] for c in range(3)]
        bu = [bu_v[0, sl] for sl in sls]

        def point(g, c2):
            xv = x_v[pl.ds(3 * g, 16)]
            x0, x1, x2 = xv[0], xv[1], xv[2]
            for blk in range(2):
                q0, q1 = 2 * blk, 2 * blk + 1
                pg = [x0 * wp[0][q] + x1 * wp[1][q] + x2 * wp[2][q]
                      for q in (q0, q1)]
                p_v[g, pl.ds(blk * 32, 32)] = plsc.pack(
                    pg[0], pg[1], format=plsc.PackFormat.INTERLEAVED)
                ug = [x0 * wu[0][q] + x1 * wu[1][q] + x2 * wu[2][q] + bu[q]
                      for q in (q0, q1)]
                u_v[g, pl.ds(blk * 32, 32)] = plsc.pack(
                    ug[0], ug[1], format=plsc.PackFormat.INTERLEAVED)
            return c2

        lax.fori_loop(0, PPW, point, 0)
        pltpu.sync_copy(p_v, p_hbm.at[pl.ds(base, PPW)])
        pltpu.sync_copy(u_v, u_hbm.at[pl.ds(base, PPW)])

    return pl.kernel(
        body,
        out_type=[jax.ShapeDtypeStruct((TOT, 64), BF),
                  jax.ShapeDtypeStruct((TOT, 64), BF)],
        mesh=mesh,
        scratch_types=scratch,
        compiler_params=pltpu.CompilerParams(
            use_tc_tiling_on_sc=False, needs_layout_passes=False),
    )


def _mm_kernel(x_ref, wt_ref, b_ref, o_ref):
    z = jnp.dot(x_ref[...], wt_ref[...],
                preferred_element_type=jnp.float32) + b_ref[...]
    o_ref[...] = jnp.maximum(z, 0.2 * z).astype(BF)


def _final_kernel(h0_ref, h1_ref, h2_ref, h3_ref, wt_ref, b_ref, o_ref):
    cat = jnp.concatenate(
        [h0_ref[...], h1_ref[...], h2_ref[...], h3_ref[...]], axis=1)
    z = jnp.dot(cat, wt_ref[...],
                preferred_element_type=jnp.float32) + b_ref[...]
    z = jnp.maximum(z, 0.2 * z)
    b = pl.program_id(0)
    o_ref[pl.ds(b, 1), :] = jnp.max(z, axis=0, keepdims=True)


def _fold(W, b, g, beta):
    return g[:, None] * W, g * b + beta


_R = 2048


def _half(xh, gidx64, gidx128, WpT, WuT, beT, W1t, b1r, W2t, b2r,
          W3t, b3r, WfT, bfr):
    f32 = jnp.float32
    grid = (TOT // _R,)
    p, u = _make_pu()(xh, WpT, WuT, beT)

    h0 = _make_pool(64, True, 64, 128)(p, gidx64, u)

    def mm(h, Wt, br, Cin, Cout):
        return pl.pallas_call(
            _mm_kernel,
            grid=grid,
            in_specs=[
                pl.BlockSpec((_R, Cin), lambda i: (i, 0)),
                pl.BlockSpec((Cin, Cout), lambda i: (0, 0)),
                pl.BlockSpec((1, Cout), lambda i: (0, 0)),
            ],
            out_specs=pl.BlockSpec((_R, Cout), lambda i: (i, 0)),
            out_shape=jax.ShapeDtypeStruct((TOT, Cout), BF),
        )(h, Wt, br)

    pool64 = _make_pool(64, False, 64, 128)
    h1 = mm(pool64(h0, gidx64), W1t, b1r, 64, 64)
    h2 = mm(pool64(h1, gidx64), W2t, b2r, 64, 128)
    h3 = mm(_make_pool(128, False, 32, 128)(h2, gidx128), W3t, b3r, 128, 256)

    return pl.pallas_call(
        _final_kernel,
        grid=(BH,),
        in_specs=[
            pl.BlockSpec((N, 64), lambda b: (b, 0)),
            pl.BlockSpec((N, 64), lambda b: (b, 0)),
            pl.BlockSpec((N, 128), lambda b: (b, 0)),
            pl.BlockSpec((N, 256), lambda b: (b, 0)),
            pl.BlockSpec((512, 512), lambda b: (0, 0)),
            pl.BlockSpec((1, 512), lambda b: (0, 0)),
        ],
        out_specs=pl.BlockSpec((BH, 512), lambda b: (0, 0)),
        out_shape=jax.ShapeDtypeStruct((BH, 512), f32),
    )(h0, h1, h2, h3, WfT, bfr)


def kernel(x, indices, W_edge, b_edge, g_edge, beta_edge,
           W1, b1, g1, beta1, W2, b2, g2, beta2, W3, b3, g3, beta3, Wf, bf):
    f32 = jnp.float32
    We, be = _fold(W_edge, b_edge, g_edge, beta_edge)
    Wc, Wd = We[:, :3], We[:, 3:]
    W1f, b1f = _fold(W1, b1, g1, beta1)
    W2f, b2f = _fold(W2, b2, g2, beta2)
    W3f, b3f = _fold(W3, b3, g3, beta3)

    xf = x.reshape(B * N, 3).astype(f32)
    idx32 = indices.astype(jnp.int32)
    off = (jnp.arange(BH, dtype=jnp.int32) * N)[:, None, None]
    import numpy as _np
    perm = _np.concatenate(
        [_np.concatenate([_np.arange(0, 32, 2), _np.arange(1, 32, 2)])
         + 32 * blk for blk in range(2)])
    halves = []
    wargs = (Wd.T[:, perm], (Wc - Wd).T[:, perm], be[perm].reshape(1, 64),
             W1f.T.astype(BF), b1f.reshape(1, 64),
             W2f.T.astype(BF), b2f.reshape(1, 128),
             W3f.T.astype(BF), b3f.reshape(1, 256),
             Wf.T.astype(BF), bf.reshape(1, 512))
    for h in range(2):
        gidx = idx32[h * BH:(h + 1) * BH] + off
        gidx64 = gidx.reshape(NW * (PPW // 64), 10, 128)
        gidx128 = gidx.reshape(NW * (PPW // 32), 5, 128)
        xh = xf.reshape(B * N * 3)[h * TOT * 3:(h + 1) * TOT * 3]
        halves.append(_half(xh, gidx64, gidx128, *wargs))
    return jnp.concatenate(halves, axis=0)


# speedup vs baseline: 106.5763x; 1.0120x over previous
import functools

import jax
import jax.numpy as jnp
from jax import lax
from jax.experimental import pallas as pl
from jax.experimental.pallas import tpu as pltpu
from jax.experimental.pallas import tpu_sc as plsc

B, N, K = 16, 2048, 20
BH = B // 2
TOT = BH * N
NC, NS = 2, 16
NW = NC * NS
PPW = TOT // NW
BF = jnp.bfloat16


def _make_pool(C, fuse_u, G, IDXW):
    ROWS = G * K
    NDMA = ROWS // IDXW
    NCHUNK = PPW // G
    assert ROWS % IDXW == 0 and NCHUNK % 2 == 0
    mesh = plsc.VectorSubcoreMesh(
        core_axis_name="c", subcore_axis_name="s",
        num_cores=NC, num_subcores=NS)
    scratch = [
        [pltpu.VMEM((NDMA, IDXW), jnp.int32) for _ in range(2)],
        [pltpu.VMEM((ROWS, C), BF) for _ in range(2)],
        [pltpu.VMEM((G, C), BF) for _ in range(2)],
        [pltpu.SemaphoreType.DMA for _ in range(2)],
    ]
    if fuse_u:
        scratch.append([pltpu.VMEM((G, C), BF) for _ in range(2)])
        scratch.append([pltpu.SemaphoreType.DMA for _ in range(2)])
    scratch.append([pltpu.SemaphoreType.DMA for _ in range(2)])

    def body(table_hbm, gidx_hbm, *rest):
        if fuse_u:
            (u_hbm, out_hbm, idx_b, rows_b, out_b, osem_b, u_b, usem_b,
             sem_b) = rest
        else:
            out_hbm, idx_b, rows_b, out_b, osem_b, sem_b = rest
        wid = lax.axis_index("s") * NC + lax.axis_index("c")
        base = wid * PPW

        def fire(c, s):
            pltpu.sync_copy(gidx_hbm.at[wid * NCHUNK + c], idx_b[s])
            for j in range(NDMA):
                pltpu.async_copy(table_hbm.at[idx_b[s].at[j]],
                                 rows_b[s].at[pl.ds(j * IDXW, IDXW)],
                                 sem_b[s])
            if fuse_u:
                pltpu.async_copy(u_hbm.at[pl.ds(base + c * G, G)], u_b[s],
                                 usem_b[s])

        def drain_compute(c, s):
            for j in range(NDMA):
                pltpu.make_async_copy(
                    table_hbm.at[idx_b[s].at[j]],
                    rows_b[s].at[pl.ds(j * IDXW, IDXW)],
                    sem_b[s]).wait()
            if fuse_u:
                pltpu.make_async_copy(u_hbm.at[pl.ds(base + c * G, G)],
                                      u_b[s], usem_b[s]).wait()

            @pl.when(c >= 2)
            def _():
                pltpu.make_async_copy(
                    out_b[s], out_hbm.at[pl.ds(base + (c - 2) * G, G)],
                    osem_b[s]).wait()

            rows_v = rows_b[s]
            out_v = out_b[s]

            def point(g, c2):
                r0 = g * K
                for co in range(C // 32):
                    sl = pl.ds(co * 32, 32)
                    acc = rows_v[r0, sl]
                    for kk in range(1, K):
                        acc = jnp.maximum(acc, rows_v[r0 + kk, sl])
                    if fuse_u:
                        t = acc + u_b[s][g, sl]
                        acc = jnp.maximum(t, BF(0.2) * t)
                    out_v[g, sl] = acc
                return c2

            lax.fori_loop(0, G, point, 0)
            pltpu.async_copy(out_v, out_hbm.at[pl.ds(base + c * G, G)],
                             osem_b[s])

        fire(0, 0)

        def pair(t, carry):
            i0 = 2 * t
            fire(i0 + 1, 1)
            drain_compute(i0, 0)

            @pl.when(i0 + 2 < NCHUNK)
            def _():
                fire(i0 + 2, 0)

            drain_compute(i0 + 1, 1)
            return carry

        lax.fori_loop(0, NCHUNK // 2, pair, 0)
        for s in range(2):
            pltpu.make_async_copy(
                out_b[s], out_hbm.at[pl.ds(base + (NCHUNK - 2 + s) * G, G)],
                osem_b[s]).wait()

    return pl.kernel(
        body,
        out_type=jax.ShapeDtypeStruct((TOT, C), BF),
        mesh=mesh,
        scratch_types=scratch,
        compiler_params=pltpu.CompilerParams(use_tc_tiling_on_sc=False),
    )


def _make_pu():
    mesh = plsc.VectorSubcoreMesh(
        core_axis_name="c", subcore_axis_name="s",
        num_cores=NC, num_subcores=NS)
    scratch = [
        pltpu.VMEM((PPW * 3 + 16,), jnp.float32),
        pltpu.VMEM((3, 64), jnp.float32),
        pltpu.VMEM((3, 64), jnp.float32),
        pltpu.VMEM((1, 64), jnp.float32),
        pltpu.VMEM((PPW, 64), BF),
        pltpu.VMEM((PPW, 64), BF),
    ]

    def body(x_hbm, wp_hbm, wu_hbm, bu_hbm, p_hbm, u_hbm,
             x_v, wp_v, wu_v, bu_v, p_v, u_v):
        wid = lax.axis_index("s") * NC + lax.axis_index("c")
        base = wid * PPW
        pltpu.sync_copy(x_hbm.at[pl.ds(base * 3, PPW * 3)],
                        x_v.at[pl.ds(0, PPW * 3)])
        pltpu.sync_copy(wp_hbm, wp_v)
        pltpu.sync_copy(wu_hbm, wu_v)
        pltpu.sync_copy(bu_hbm, bu_v)

        sls = [pl.ds(q * 16, 16) for q in range(4)]
        wp = [[wp_v[c, sl] for sl in sls] for c in range(3)]
        wu = [[wu_v[c, sl] for sl in sls] for c in range(3)]
        bu = [bu_v[0, sl] for sl in sls]

        def point(g, c2):
            xv = x_v[pl.ds(3 * g, 16)]
            x0, x1, x2 = xv[0], xv[1], xv[2]
            for blk in range(2):
                q0, q1 = 2 * blk, 2 * blk + 1
                pg = [x0 * wp[0][q] + x1 * wp[1][q] + x2 * wp[2][q]
                      for q in (q0, q1)]
                p_v[g, pl.ds(blk * 32, 32)] = plsc.pack(
                    pg[0], pg[1], format=plsc.PackFormat.INTERLEAVED)
                ug = [x0 * wu[0][q] + x1 * wu[1][q] + x2 * wu[2][q] + bu[q]
                      for q in (q0, q1)]
                u_v[g, pl.ds(blk * 32, 32)] = plsc.pack(
                    ug[0], ug[1], format=plsc.PackFormat.INTERLEAVED)
            return c2

        lax.fori_loop(0, PPW, point, 0)
        pltpu.sync_copy(p_v, p_hbm.at[pl.ds(base, PPW)])
        pltpu.sync_copy(u_v, u_hbm.at[pl.ds(base, PPW)])

    return pl.kernel(
        body,
        out_type=[jax.ShapeDtypeStruct((TOT, 64), BF),
                  jax.ShapeDtypeStruct((TOT, 64), BF)],
        mesh=mesh,
        scratch_types=scratch,
        compiler_params=pltpu.CompilerParams(
            use_tc_tiling_on_sc=False, needs_layout_passes=False),
    )


def _mm_kernel(x_ref, wt_ref, b_ref, o_ref):
    z = jnp.dot(x_ref[...], wt_ref[...],
                preferred_element_type=jnp.float32) + b_ref[...]
    o_ref[...] = jnp.maximum(z, 0.2 * z).astype(BF)


def _final_kernel(h0_ref, h1_ref, h2_ref, h3_ref, wt_ref, b_ref, o_ref):
    cat = jnp.concatenate(
        [h0_ref[...], h1_ref[...], h2_ref[...], h3_ref[...]], axis=1)
    z = jnp.dot(cat, wt_ref[...],
                preferred_element_type=jnp.float32) + b_ref[...]
    z = jnp.maximum(z, 0.2 * z)
    b = pl.program_id(0)
    o_ref[pl.ds(b, 1), :] = jnp.max(z, axis=0, keepdims=True)


def _fold(W, b, g, beta):
    return g[:, None] * W, g * b + beta


_R = 2048


def _half(xh, gidx64, gidx128, WpT, WuT, beT, W1t, b1r, W2t, b2r,
          W3t, b3r, WfT, bfr):
    f32 = jnp.float32
    grid = (TOT // _R,)
    p, u = _make_pu()(xh, WpT, WuT, beT)

    h0 = _make_pool(64, True, 64, 128)(p, gidx64, u)

    def mm(h, Wt, br, Cin, Cout):
        return pl.pallas_call(
            _mm_kernel,
            grid=grid,
            in_specs=[
                pl.BlockSpec((_R, Cin), lambda i: (i, 0)),
                pl.BlockSpec((Cin, Cout), lambda i: (0, 0)),
                pl.BlockSpec((1, Cout), lambda i: (0, 0)),
            ],
            out_specs=pl.BlockSpec((_R, Cout), lambda i: (i, 0)),
            out_shape=jax.ShapeDtypeStruct((TOT, Cout), BF),
        )(h, Wt, br)

    pool64 = _make_pool(64, False, 64, 128)
    h1 = mm(pool64(h0, gidx64), W1t, b1r, 64, 64)
    h2 = mm(pool64(h1, gidx64), W2t, b2r, 64, 128)
    h3 = mm(_make_pool(128, False, 32, 128)(h2, gidx128), W3t, b3r, 128, 256)

    return pl.pallas_call(
        _final_kernel,
        grid=(BH,),
        in_specs=[
            pl.BlockSpec((N, 64), lambda b: (b, 0)),
            pl.BlockSpec((N, 64), lambda b: (b, 0)),
            pl.BlockSpec((N, 128), lambda b: (b, 0)),
            pl.BlockSpec((N, 256), lambda b: (b, 0)),
            pl.BlockSpec((512, 512), lambda b: (0, 0)),
            pl.BlockSpec((1, 512), lambda b: (0, 0)),
        ],
        out_specs=pl.BlockSpec((BH, 512), lambda b: (0, 0)),
        out_shape=jax.ShapeDtypeStruct((BH, 512), f32),
    )(h0, h1, h2, h3, WfT, bfr)


def kernel(x, indices, W_edge, b_edge, g_edge, beta_edge,
           W1, b1, g1, beta1, W2, b2, g2, beta2, W3, b3, g3, beta3, Wf, bf):
    f32 = jnp.float32
    We, be = _fold(W_edge, b_edge, g_edge, beta_edge)
    Wc, Wd = We[:, :3], We[:, 3:]
    W1f, b1f = _fold(W1, b1, g1, beta1)
    W2f, b2f = _fold(W2, b2, g2, beta2)
    W3f, b3f = _fold(W3, b3, g3, beta3)

    xf = x.reshape(B * N, 3).astype(f32)
    idx32 = indices.astype(jnp.int32)
    off = (jnp.arange(BH, dtype=jnp.int32) * N)[:, None, None]
    import numpy as _np
    perm = _np.concatenate(
        [_np.concatenate([_np.arange(0, 32, 2), _np.arange(1, 32, 2)])
         + 32 * blk for blk in range(2)])
    halves = []
    wargs = (Wd.T[:, perm], (Wc - Wd).T[:, perm], be[perm].reshape(1, 64),
             W1f.T.astype(BF), b1f.reshape(1, 64),
             W2f.T.astype(BF), b2f.reshape(1, 128),
             W3f.T.astype(BF), b3f.reshape(1, 256),
             Wf.T.astype(BF), bf.reshape(1, 512))
    for h in range(2):
        gidx = idx32[h * BH:(h + 1) * BH] + off
        gidx64 = gidx.reshape(NW * (PPW // 64), 10, 128)
        gidx128 = gidx.reshape(NW * (PPW // 32), 5, 128)
        xh = xf.reshape(B * N * 3)[h * TOT * 3:(h + 1) * TOT * 3]
        halves.append(_half(xh, gidx64, gidx128, *wargs))
    return jnp.concatenate(halves, axis=0)
